# baseline (speedup 1.0000x reference)
import jax
import jax.numpy as jnp
from jax import lax
from jax.experimental import pallas as pl
from jax.experimental.pallas import tpu as pltpu
from jax.experimental.pallas import tpu_sc as plsc

_B, _S, _D = 16384, 200, 64
_N2 = _B * _S // 2

_INFO = plsc.get_sparse_core_info()
_NC, _NS = _INFO.num_cores, _INFO.num_subcores
_NW = _NC * _NS
_PER_W = _N2 // _NW
_C = 512
_CHUNKS = _PER_W // _C


def _sc_body(idx_hbm, tab_hbm, out_hbm, idx_v, rows_v, sem):
    wid = lax.axis_index("s") * _NC + lax.axis_index("c")
    base = wid * _PER_W

    def step(i, _):
        off = base + i * _C
        pltpu.sync_copy(idx_hbm.at[pl.ds(off, _C)], idx_v)
        pltpu.async_copy(tab_hbm.at[idx_v], rows_v, sem).wait()
        pltpu.sync_copy(rows_v, out_hbm.at[pl.ds(off, _C)])
        return ()

    lax.fori_loop(0, _CHUNKS, step, ())


def kernel(inputs, table):
    idx2 = inputs.astype(jnp.int32).reshape(_N2, 2)
    cidx = 7 * idx2[:, 0] + idx2[:, 1]
    tab49 = jnp.concatenate(
        [jnp.repeat(table, 7, axis=0), jnp.tile(table, (7, 1))], axis=1)
    out = pl.kernel(
        _sc_body,
        out_type=jax.ShapeDtypeStruct((_N2, 2 * _D), jnp.float32),
        mesh=plsc.VectorSubcoreMesh(core_axis_name="c", subcore_axis_name="s"),
        scratch_types=[
            pltpu.VMEM((_C,), jnp.int32),
            pltpu.VMEM((_C, 2 * _D), jnp.float32),
            pltpu.SemaphoreType.DMA,
        ],
    )(cidx, tab49)
    return out.reshape(_B, _S, _D)
